# Initial kernel scaffold; baseline (speedup 1.0000x reference)
import jax
import jax.numpy as jnp
from jax.experimental import pallas as pl


def kernel(in_feat, edge_index, W1, b1, W2, b2, W3, b3):
    raise NotImplementedError("write your pallas kernel here")


# speedup vs baseline: 4.8597x; 4.8597x over previous
import functools

import jax
import jax.numpy as jnp
from jax import lax
from jax.experimental import pallas as pl
from jax.experimental.pallas import tpu as pltpu
from jax.experimental.pallas import tpu_sc as plsc

N = 10000
E = 320000
NC = 2
NS = 16
NW = NC * NS
EW = E // NW
CH = 128
NFULL = EW // CH
REM = EW - NFULL * CH
NPAD = 10240
RPT = NPAD // NS
WCH = 128
NWCH = RPT // WCH


def _mesh():
    return plsc.VectorSubcoreMesh(core_axis_name="c", subcore_axis_name="s")


def _zero_rows(ref, nrows, ncols):
    z = jnp.zeros((16,), jnp.float32)

    def body(i, _):
        for k in range(ncols // 16):
            ref[i, pl.ds(k * 16, 16)] = z
        return 0

    lax.fori_loop(0, nrows, body, 0)


def _fill_ones(ref, nrows, ncols):
    o = jnp.full((16,), 1.0, jnp.float32)

    def body(i, _):
        for k in range(ncols // 16):
            ref[i, pl.ds(k * 16, 16)] = o
        return 0

    lax.fori_loop(0, nrows, body, 0)


@functools.partial(
    pl.kernel,
    out_type=(
        jax.ShapeDtypeStruct((NC, NPAD, 128), jnp.float32),
        jax.ShapeDtypeStruct((NC, NPAD, 128), jnp.float32),
    ),
    mesh=_mesh(),
    scratch_types=[
        pltpu.VMEM((CH,), jnp.int32),
        pltpu.VMEM((REM,), jnp.int32),
        pltpu.VMEM((CH, 128), jnp.float32),
        pltpu.VMEM((REM, 128), jnp.float32),
        pltpu.VMEM((WCH, 128), jnp.float32),
        pltpu.VMEM_SHARED((NPAD, 128), jnp.float32),
    ],
)
def _deg(src_hbm, dst_hbm, out_src, out_dst, ia, ia16,
         ones_v, ones16_v, stage_v, sh):
    cc = lax.axis_index("c")
    ss = lax.axis_index("s")
    wid = ss * NC + cc
    base = wid * EW

    _fill_ones(ones_v, CH, 128)
    _fill_ones(ones16_v, REM, 128)

    for idx_hbm, out in ((src_hbm, out_src), (dst_hbm, out_dst)):
        _zero_rows(stage_v, WCH, 128)
        for j in range(NWCH):
            r0 = ss * RPT + j * WCH
            pltpu.sync_copy(stage_v, sh.at[pl.ds(r0, WCH)])
        plsc.subcore_barrier()

        def body(c, _):
            off = base + c * CH
            pltpu.sync_copy(idx_hbm.at[pl.ds(off, CH)], ia)
            pltpu.sync_copy(ones_v, sh.at[ia], add=True)
            return 0

        lax.fori_loop(0, NFULL, body, 0)
        off = base + NFULL * CH
        pltpu.sync_copy(idx_hbm.at[pl.ds(off, REM)], ia16)
        pltpu.sync_copy(ones16_v, sh.at[ia16], add=True)

        plsc.subcore_barrier()
        for j in range(NWCH):
            r0 = ss * RPT + j * WCH
            pltpu.sync_copy(sh.at[pl.ds(r0, WCH)], stage_v)
            pltpu.sync_copy(stage_v, out.at[cc, pl.ds(r0, WCH)])
        plsc.subcore_barrier()


def _make_agg(F):
    @functools.partial(
        pl.kernel,
        out_type=jax.ShapeDtypeStruct((NC, NPAD, F), jnp.float32),
        mesh=_mesh(),
        scratch_types=[
            pltpu.VMEM((CH,), jnp.int32),
            pltpu.VMEM((CH,), jnp.int32),
            pltpu.VMEM((REM,), jnp.int32),
            pltpu.VMEM((REM,), jnp.int32),
            pltpu.VMEM((CH, F), jnp.float32),
            pltpu.VMEM((REM, F), jnp.float32),
            pltpu.VMEM((WCH, F), jnp.float32),
            pltpu.VMEM_SHARED((NPAD, F), jnp.float32),
            pltpu.SemaphoreType.DMA,
        ],
    )
    def agg(t_hbm, src_hbm, dst_hbm, out, isrc, idst, isrc16, idst16,
            rows_v, rows16_v, stage_v, sh_agg, sem):
        cc = lax.axis_index("c")
        ss = lax.axis_index("s")
        wid = ss * NC + cc
        base = wid * EW

        _zero_rows(stage_v, WCH, F)
        for j in range(NWCH):
            r0 = ss * RPT + j * WCH
            pltpu.sync_copy(stage_v, sh_agg.at[pl.ds(r0, WCH)])
        plsc.subcore_barrier()

        def body(c, _):
            off = base + c * CH
            pltpu.sync_copy(src_hbm.at[pl.ds(off, CH)], isrc)
            pltpu.sync_copy(dst_hbm.at[pl.ds(off, CH)], idst)
            pltpu.async_copy(t_hbm.at[isrc], rows_v, sem).wait()
            pltpu.sync_copy(rows_v, sh_agg.at[idst], add=True)
            return 0

        lax.fori_loop(0, NFULL, body, 0)
        off = base + NFULL * CH
        pltpu.sync_copy(src_hbm.at[pl.ds(off, REM)], isrc16)
        pltpu.sync_copy(dst_hbm.at[pl.ds(off, REM)], idst16)
        pltpu.async_copy(t_hbm.at[isrc16], rows16_v, sem).wait()
        pltpu.sync_copy(rows16_v, sh_agg.at[idst16], add=True)

        plsc.subcore_barrier()
        for j in range(NWCH):
            r0 = ss * RPT + j * WCH
            pltpu.sync_copy(sh_agg.at[pl.ds(r0, WCH)], stage_v)
            pltpu.sync_copy(stage_v, out.at[cc, pl.ds(r0, WCH)])

    return agg


_agg128 = _make_agg(128)


BR = 1000


def _scale_from_counts(c_ref):
    c = c_ref[0, :, 0:1] + c_ref[1, :, 0:1]
    return lax.rsqrt(jnp.maximum(c, 1.0))


def _tc_first(x, cnt_src, W1):
    def body(x_ref, cs_ref, w_ref, o_ref):
        so = _scale_from_counts(cs_ref)
        o_ref[...] = jnp.dot(x_ref[...] * so, w_ref[...],
                             preferred_element_type=jnp.float32)

    return pl.pallas_call(
        body,
        grid=(N // BR,),
        in_specs=[
            pl.BlockSpec((BR, 128), lambda i: (i, 0)),
            pl.BlockSpec((2, BR, 128), lambda i: (0, i, 0)),
            pl.BlockSpec((128, 128), lambda i: (0, 0)),
        ],
        out_specs=pl.BlockSpec((BR, 128), lambda i: (i, 0)),
        out_shape=jax.ShapeDtypeStruct((N, 128), jnp.float32),
    )(x, cnt_src, W1)


def _tc_mid(parts, cnt_dst, cnt_src, b, Wn, F, Fn):
    def body(p_ref, cd_ref, cs_ref, b_ref, w_ref, o_ref, t_ref):
        si = _scale_from_counts(cd_ref)
        o = (p_ref[0] + p_ref[1]) * si + b_ref[...]
        o_ref[...] = o
        so = _scale_from_counts(cs_ref)
        x = jnp.maximum(o, 0.0) * so
        t_ref[...] = jnp.dot(x, w_ref[...], preferred_element_type=jnp.float32)

    return pl.pallas_call(
        body,
        grid=(N // BR,),
        in_specs=[
            pl.BlockSpec((2, BR, F), lambda i: (0, i, 0)),
            pl.BlockSpec((2, BR, 128), lambda i: (0, i, 0)),
            pl.BlockSpec((2, BR, 128), lambda i: (0, i, 0)),
            pl.BlockSpec((1, F), lambda i: (0, 0)),
            pl.BlockSpec((F, Fn), lambda i: (0, 0)),
        ],
        out_specs=(
            pl.BlockSpec((BR, F), lambda i: (i, 0)),
            pl.BlockSpec((BR, Fn), lambda i: (i, 0)),
        ),
        out_shape=(
            jax.ShapeDtypeStruct((N, F), jnp.float32),
            jax.ShapeDtypeStruct((N, Fn), jnp.float32),
        ),
    )(parts, cnt_dst, cnt_src, b, Wn)


def _tc_last(parts, cnt_dst, b, F):
    def body(p_ref, cd_ref, b_ref, o_ref):
        si = _scale_from_counts(cd_ref)
        o_ref[...] = (p_ref[0, :, :F] + p_ref[1, :, :F]) * si + b_ref[...]

    return pl.pallas_call(
        body,
        grid=(N // BR,),
        in_specs=[
            pl.BlockSpec((2, BR, 128), lambda i: (0, i, 0)),
            pl.BlockSpec((2, BR, 128), lambda i: (0, i, 0)),
            pl.BlockSpec((1, F), lambda i: (0, 0)),
        ],
        out_specs=pl.BlockSpec((BR, F), lambda i: (i, 0)),
        out_shape=jax.ShapeDtypeStruct((N, F), jnp.float32),
    )(parts, cnt_dst, b)


def kernel(in_feat, edge_index, W1, b1, W2, b2, W3, b3):
    src = edge_index[0].astype(jnp.int32)
    dst = edge_index[1].astype(jnp.int32)
    cnt_src, cnt_dst = _deg(src, dst)
    t1 = _tc_first(in_feat, cnt_src, W1)
    p1 = _agg128(t1, src, dst)
    _, t2 = _tc_mid(p1, cnt_dst, cnt_src, b1.reshape(1, -1), W2, 128, 128)
    p2 = _agg128(t2, src, dst)
    W3p = jnp.pad(W3, ((0, 0), (0, 128 - W3.shape[1])))
    embed, t3 = _tc_mid(p2, cnt_dst, cnt_src, b2.reshape(1, -1), W3p, 128, 128)
    p3 = _agg128(t3, src, dst)
    h = _tc_last(p3, cnt_dst, b3.reshape(1, -1), 64)
    return (embed, h)


# speedup vs baseline: 6.8278x; 1.4050x over previous
import functools

import jax
import jax.numpy as jnp
from jax import lax
from jax.experimental import pallas as pl
from jax.experimental.pallas import tpu as pltpu
from jax.experimental.pallas import tpu_sc as plsc

N = 10000
E = 320000
NC = 2
NS = 16
NW = NC * NS
EW = E // NW
CH = 128
NFULL = EW // CH
REM = EW - NFULL * CH
NPAD = 10240
RPT = NPAD // NS
WCH = 128
NWCH = RPT // WCH


def _mesh():
    return plsc.VectorSubcoreMesh(core_axis_name="c", subcore_axis_name="s")


def _zero_rows(ref, nrows, ncols):
    z = jnp.zeros((16,), jnp.float32)

    def body(i, _):
        for k in range(ncols // 16):
            ref[i, pl.ds(k * 16, 16)] = z
        return 0

    lax.fori_loop(0, nrows, body, 0)


def _fill_ones(ref, nrows, ncols):
    o = jnp.full((16,), 1.0, jnp.float32)

    def body(i, _):
        for k in range(ncols // 16):
            ref[i, pl.ds(k * 16, 16)] = o
        return 0

    lax.fori_loop(0, nrows, body, 0)


@functools.partial(
    pl.kernel,
    out_type=(
        jax.ShapeDtypeStruct((NC, NPAD, 128), jnp.float32),
        jax.ShapeDtypeStruct((NC, NPAD, 128), jnp.float32),
    ),
    mesh=_mesh(),
    scratch_types=[
        pltpu.VMEM((CH,), jnp.int32),
        pltpu.VMEM((REM,), jnp.int32),
        pltpu.VMEM((CH, 128), jnp.float32),
        pltpu.VMEM((REM, 128), jnp.float32),
        pltpu.VMEM((WCH, 128), jnp.float32),
        pltpu.VMEM_SHARED((NPAD, 128), jnp.float32),
    ],
)
def _deg(src_hbm, dst_hbm, out_src, out_dst, ia, ia16,
         ones_v, ones16_v, stage_v, sh):
    cc = lax.axis_index("c")
    ss = lax.axis_index("s")
    wid = ss * NC + cc
    base = wid * EW

    _fill_ones(ones_v, CH, 128)
    _fill_ones(ones16_v, REM, 128)

    for idx_hbm, out in ((src_hbm, out_src), (dst_hbm, out_dst)):
        _zero_rows(stage_v, WCH, 128)
        for j in range(NWCH):
            r0 = ss * RPT + j * WCH
            pltpu.sync_copy(stage_v, sh.at[pl.ds(r0, WCH)])
        plsc.subcore_barrier()

        def body(c, _):
            off = base + c * CH
            pltpu.sync_copy(idx_hbm.at[pl.ds(off, CH)], ia)
            pltpu.sync_copy(ones_v, sh.at[ia], add=True)
            return 0

        lax.fori_loop(0, NFULL, body, 0)
        off = base + NFULL * CH
        pltpu.sync_copy(idx_hbm.at[pl.ds(off, REM)], ia16)
        pltpu.sync_copy(ones16_v, sh.at[ia16], add=True)

        plsc.subcore_barrier()
        for j in range(NWCH):
            r0 = ss * RPT + j * WCH
            pltpu.sync_copy(sh.at[pl.ds(r0, WCH)], stage_v)
            pltpu.sync_copy(stage_v, out.at[cc, pl.ds(r0, WCH)])
        plsc.subcore_barrier()


NH = NFULL // 2


def _make_agg(F):
    @functools.partial(
        pl.kernel,
        out_type=jax.ShapeDtypeStruct((NC, NPAD, F), jnp.float32),
        mesh=_mesh(),
        scratch_types=[
            pltpu.VMEM((CH,), jnp.int32),
            pltpu.VMEM((CH,), jnp.int32),
            pltpu.VMEM((CH,), jnp.int32),
            pltpu.VMEM((CH,), jnp.int32),
            pltpu.VMEM((REM,), jnp.int32),
            pltpu.VMEM((REM,), jnp.int32),
            pltpu.VMEM((CH, F), jnp.float32),
            pltpu.VMEM((CH, F), jnp.float32),
            pltpu.VMEM((REM, F), jnp.float32),
            pltpu.VMEM_SHARED((NPAD, F), jnp.float32),
            pltpu.SemaphoreType.DMA,
            pltpu.SemaphoreType.DMA,
        ],
    )
    def agg(t_hbm, src_hbm, dst_hbm, out, ia0, ia1, ib0, ib1, isrc16, idst16,
            rows0, rows1, rows16, sh_agg, sem0, sem1):
        cc = lax.axis_index("c")
        ss = lax.axis_index("s")
        wid = ss * NC + cc
        base = wid * EW

        _zero_rows(rows0, WCH, F)
        for j in range(NWCH):
            r0 = ss * RPT + j * WCH
            pltpu.sync_copy(rows0, sh_agg.at[pl.ds(r0, WCH)])
        plsc.subcore_barrier()

        pltpu.sync_copy(src_hbm.at[pl.ds(base, CH)], ia0)
        pltpu.async_copy(t_hbm.at[ia0], rows0, sem0)

        def body(kk, _):
            e = base + (2 * kk) * CH
            o = e + CH
            pltpu.sync_copy(src_hbm.at[pl.ds(o, CH)], ia1)
            pltpu.async_copy(t_hbm.at[ia1], rows1, sem1)
            pltpu.make_async_copy(t_hbm.at[ia0], rows0, sem0).wait()
            pltpu.sync_copy(dst_hbm.at[pl.ds(e, CH)], ib0)
            pltpu.sync_copy(rows0, sh_agg.at[ib0], add=True)
            ne = e + 2 * CH
            pltpu.sync_copy(src_hbm.at[pl.ds(ne, CH)], ia0)
            pltpu.async_copy(t_hbm.at[ia0], rows0, sem0)
            pltpu.make_async_copy(t_hbm.at[ia1], rows1, sem1).wait()
            pltpu.sync_copy(dst_hbm.at[pl.ds(o, CH)], ib1)
            pltpu.sync_copy(rows1, sh_agg.at[ib1], add=True)
            return 0

        lax.fori_loop(0, NH - 1, body, 0)

        e = base + (NFULL - 2) * CH
        o = e + CH
        pltpu.sync_copy(src_hbm.at[pl.ds(o, CH)], ia1)
        pltpu.async_copy(t_hbm.at[ia1], rows1, sem1)
        pltpu.make_async_copy(t_hbm.at[ia0], rows0, sem0).wait()
        pltpu.sync_copy(dst_hbm.at[pl.ds(e, CH)], ib0)
        pltpu.sync_copy(rows0, sh_agg.at[ib0], add=True)
        pltpu.make_async_copy(t_hbm.at[ia1], rows1, sem1).wait()
        pltpu.sync_copy(dst_hbm.at[pl.ds(o, CH)], ib1)
        pltpu.sync_copy(rows1, sh_agg.at[ib1], add=True)

        off = base + NFULL * CH
        pltpu.sync_copy(src_hbm.at[pl.ds(off, REM)], isrc16)
        pltpu.sync_copy(dst_hbm.at[pl.ds(off, REM)], idst16)
        pltpu.async_copy(t_hbm.at[isrc16], rows16, sem0).wait()
        pltpu.sync_copy(rows16, sh_agg.at[idst16], add=True)

        plsc.subcore_barrier()
        for j in range(NWCH):
            r0 = ss * RPT + j * WCH
            pltpu.sync_copy(sh_agg.at[pl.ds(r0, WCH)], rows0)
            pltpu.sync_copy(rows0, out.at[cc, pl.ds(r0, WCH)])

    return agg


_agg128 = _make_agg(128)


BR = 1000


def _scale_from_counts(c_ref):
    c = c_ref[0, :, 0:1] + c_ref[1, :, 0:1]
    return lax.rsqrt(jnp.maximum(c, 1.0))


def _tc_first(x, cnt_src, W1):
    def body(x_ref, cs_ref, w_ref, o_ref):
        so = _scale_from_counts(cs_ref)
        o_ref[...] = jnp.dot(x_ref[...] * so, w_ref[...],
                             preferred_element_type=jnp.float32)

    return pl.pallas_call(
        body,
        grid=(N // BR,),
        in_specs=[
            pl.BlockSpec((BR, 128), lambda i: (i, 0)),
            pl.BlockSpec((2, BR, 128), lambda i: (0, i, 0)),
            pl.BlockSpec((128, 128), lambda i: (0, 0)),
        ],
        out_specs=pl.BlockSpec((BR, 128), lambda i: (i, 0)),
        out_shape=jax.ShapeDtypeStruct((N, 128), jnp.float32),
    )(x, cnt_src, W1)


def _tc_mid(parts, cnt_dst, cnt_src, b, Wn, F, Fn):
    def body(p_ref, cd_ref, cs_ref, b_ref, w_ref, o_ref, t_ref):
        si = _scale_from_counts(cd_ref)
        o = (p_ref[0] + p_ref[1]) * si + b_ref[...]
        o_ref[...] = o
        so = _scale_from_counts(cs_ref)
        x = jnp.maximum(o, 0.0) * so
        t_ref[...] = jnp.dot(x, w_ref[...], preferred_element_type=jnp.float32)

    return pl.pallas_call(
        body,
        grid=(N // BR,),
        in_specs=[
            pl.BlockSpec((2, BR, F), lambda i: (0, i, 0)),
            pl.BlockSpec((2, BR, 128), lambda i: (0, i, 0)),
            pl.BlockSpec((2, BR, 128), lambda i: (0, i, 0)),
            pl.BlockSpec((1, F), lambda i: (0, 0)),
            pl.BlockSpec((F, Fn), lambda i: (0, 0)),
        ],
        out_specs=(
            pl.BlockSpec((BR, F), lambda i: (i, 0)),
            pl.BlockSpec((BR, Fn), lambda i: (i, 0)),
        ),
        out_shape=(
            jax.ShapeDtypeStruct((N, F), jnp.float32),
            jax.ShapeDtypeStruct((N, Fn), jnp.float32),
        ),
    )(parts, cnt_dst, cnt_src, b, Wn)


def _tc_last(parts, cnt_dst, b, F):
    def body(p_ref, cd_ref, b_ref, o_ref):
        si = _scale_from_counts(cd_ref)
        o_ref[...] = (p_ref[0, :, :F] + p_ref[1, :, :F]) * si + b_ref[...]

    return pl.pallas_call(
        body,
        grid=(N // BR,),
        in_specs=[
            pl.BlockSpec((2, BR, 128), lambda i: (0, i, 0)),
            pl.BlockSpec((2, BR, 128), lambda i: (0, i, 0)),
            pl.BlockSpec((1, F), lambda i: (0, 0)),
        ],
        out_specs=pl.BlockSpec((BR, F), lambda i: (i, 0)),
        out_shape=jax.ShapeDtypeStruct((N, F), jnp.float32),
    )(parts, cnt_dst, b)


def kernel(in_feat, edge_index, W1, b1, W2, b2, W3, b3):
    src = edge_index[0].astype(jnp.int32)
    dst = edge_index[1].astype(jnp.int32)
    cnt_src, cnt_dst = _deg(src, dst)
    t1 = _tc_first(in_feat, cnt_src, W1)
    p1 = _agg128(t1, src, dst)
    _, t2 = _tc_mid(p1, cnt_dst, cnt_src, b1.reshape(1, -1), W2, 128, 128)
    p2 = _agg128(t2, src, dst)
    W3p = jnp.pad(W3, ((0, 0), (0, 128 - W3.shape[1])))
    embed, t3 = _tc_mid(p2, cnt_dst, cnt_src, b2.reshape(1, -1), W3p, 128, 128)
    p3 = _agg128(t3, src, dst)
    h = _tc_last(p3, cnt_dst, b3.reshape(1, -1), 64)
    return (embed, h)
